# Initial kernel scaffold; baseline (speedup 1.0000x reference)
import jax
import jax.numpy as jnp
from jax.experimental import pallas as pl


def kernel(inputs, grid_positions, embedding, topographical_embedding, x_learn, y_learn):
    raise NotImplementedError("write your pallas kernel here")


# speedup vs baseline: 2.3358x; 2.3358x over previous
import jax
import jax.numpy as jnp
from jax.experimental import pallas as pl

N_IN = 128
EMB_DIM = 512


def _embed_kernel(inp_ref, topo_ref, out_ref):
    out_ref[0, :, :N_IN] = inp_ref[0]
    out_ref[0, :, N_IN:] = topo_ref[...]


def kernel(inputs, grid_positions, embedding, topographical_embedding, x_learn, y_learn):
    B, GRID, _ = inputs.shape
    topo = topographical_embedding[:, : EMB_DIM - N_IN]

    out = pl.pallas_call(
        _embed_kernel,
        grid=(B,),
        in_specs=[
            pl.BlockSpec((1, GRID, N_IN), lambda b: (b, 0, 0)),
            pl.BlockSpec((GRID, EMB_DIM - N_IN), lambda b: (0, 0)),
        ],
        out_specs=pl.BlockSpec((1, GRID, EMB_DIM), lambda b: (b, 0, 0)),
        out_shape=jax.ShapeDtypeStruct((B, GRID, EMB_DIM), jnp.float32),
    )(inputs, topo)
    return out.reshape(B * GRID, 1, EMB_DIM)


# speedup vs baseline: 5.6027x; 2.3986x over previous
import jax
import jax.numpy as jnp
from jax.experimental import pallas as pl

N_IN = 128
EMB_DIM = 512


def _embed_kernel(inp_ref, topo_ref, out_ref):
    out_ref[:, 0, :N_IN] = inp_ref[0]
    out_ref[:, 0, N_IN:] = topo_ref[:, : EMB_DIM - N_IN]


def kernel(inputs, grid_positions, embedding, topographical_embedding, x_learn, y_learn):
    B, GRID, _ = inputs.shape

    out = pl.pallas_call(
        _embed_kernel,
        grid=(B,),
        in_specs=[
            pl.BlockSpec((1, GRID, N_IN), lambda b: (b, 0, 0)),
            pl.BlockSpec((GRID, EMB_DIM), lambda b: (0, 0)),
        ],
        out_specs=pl.BlockSpec((GRID, 1, EMB_DIM), lambda b: (b, 0, 0)),
        out_shape=jax.ShapeDtypeStruct((B * GRID, 1, EMB_DIM), jnp.float32),
    )(inputs, topographical_embedding)
    return out
